# baseline (speedup 1.0000x reference)
import functools

import jax
import jax.numpy as jnp
from jax import lax
from jax.experimental import pallas as pl
from jax.experimental.pallas import tpu as pltpu
from jax.experimental.pallas import tpu_sc as plsc

BATCH = 4096
SEQ = 50
EMBED_DIM = 64

NC = 2
NS = 16
NW = NC * NS
CHUNK = 128
N_ROWS = BATCH * SEQ
ROWS_PER_W = N_ROWS // NW
N_CHUNKS = ROWS_PER_W // CHUNK


@jax.jit
def _gather(idx, table):
    mesh = plsc.VectorSubcoreMesh(
        core_axis_name="c", subcore_axis_name="s",
        num_cores=NC, num_subcores=NS,
    )

    @functools.partial(
        pl.kernel,
        out_type=jax.ShapeDtypeStruct((N_ROWS, EMBED_DIM), jnp.float32),
        mesh=mesh,
        compiler_params=pltpu.CompilerParams(use_tc_tiling_on_sc=False),
        scratch_types=[
            pltpu.VMEM((N_CHUNKS, CHUNK), jnp.int32),
            pltpu.VMEM((2, CHUNK, EMBED_DIM), jnp.float32),
            pltpu.SemaphoreType.DMA,
            pltpu.SemaphoreType.DMA,
        ],
    )
    def k(idx_hbm, table_hbm, out_hbm, idx_v, buf_v, gsem, ssem):
        wid = lax.axis_index("s") * NC + lax.axis_index("c")
        base = wid * ROWS_PER_W
        pltpu.sync_copy(idx_hbm.at[wid], idx_v)

        pltpu.async_copy(table_hbm.at[idx_v.at[0]], buf_v.at[0], gsem)

        @pl.loop(0, N_CHUNKS, step=2)
        def _(j):
            for b in range(2):
                cur = j + b
                nxt = cur + 1

                @pl.when(nxt < N_CHUNKS)
                def _():
                    @pl.when(nxt >= 2)
                    def _():
                        pltpu.make_async_copy(
                            buf_v.at[1 - b],
                            out_hbm.at[pl.ds(base, CHUNK)],
                            ssem,
                        ).wait()

                    pltpu.async_copy(
                        table_hbm.at[idx_v.at[nxt]], buf_v.at[1 - b], gsem
                    )

                pltpu.make_async_copy(
                    table_hbm.at[idx_v.at[cur]], buf_v.at[b], gsem
                ).wait()
                pltpu.async_copy(
                    buf_v.at[b],
                    out_hbm.at[pl.ds(base + cur * CHUNK, CHUNK)],
                    ssem,
                )

        @pl.loop(0, 2)
        def _(j):
            pltpu.make_async_copy(
                buf_v.at[0], out_hbm.at[pl.ds(base, CHUNK)], ssem
            ).wait()

    return k(idx, table)


def kernel(tokens, mask, table):
    del mask
    idx = tokens.astype(jnp.int32).reshape(NW, N_CHUNKS, CHUNK)
    out = _gather(idx, table)
    return out.reshape(BATCH, SEQ, EMBED_DIM)
